# baseline (speedup 1.0000x reference)
import functools

import numpy as np
import jax
import jax.numpy as jnp
from jax import lax
from jax.experimental import pallas as pl
from jax.experimental.pallas import tpu as pltpu
from jax.experimental.pallas import tpu_sc as plsc

_NUM_CORES = 2
_NUM_SUBCORES = 16
_LANES = 16
_NW = _NUM_CORES * _NUM_SUBCORES
_IL = 4


def _u_thresholds(i_dim: int, m_dim: int) -> jax.Array:
    u = jax.random.uniform(jax.random.key(42), (i_dim, m_dim, 1),
                           dtype=jnp.float32)
    return u.reshape(i_dim * m_dim)


@functools.lru_cache(maxsize=2)
def _build_sc_call(rows: int, np1: int):
    rows_per_w = rows // _NW
    chunk = 512
    assert rows_per_w % chunk == 0 and chunk % (_LANES * _IL) == 0
    n_chunks = rows_per_w // chunk
    assert n_chunks % 2 == 0
    chunk_words = chunk * np1
    groups_per_chunk = chunk // _LANES

    mesh = plsc.VectorSubcoreMesh(core_axis_name="c", subcore_axis_name="s")

    @functools.partial(
        pl.kernel,
        out_type=jax.ShapeDtypeStruct((rows,), jnp.int32),
        mesh=mesh,
        compiler_params=pltpu.CompilerParams(needs_layout_passes=False),
        scratch_types=[
            pltpu.VMEM((chunk_words,), jnp.float32),
            pltpu.VMEM((chunk_words,), jnp.float32),
            pltpu.VMEM((rows_per_w,), jnp.float32),
            pltpu.VMEM((rows_per_w,), jnp.int32),
            pltpu.SemaphoreType.DMA,
            pltpu.SemaphoreType.DMA,
        ],
    )
    def sc_count(pi_hbm, u_hbm, out_hbm, buf_a, buf_b, u_v, out_v,
                 sem_a, sem_b):
        wid = lax.axis_index("s") * _NUM_CORES + lax.axis_index("c")
        row0 = wid * rows_per_w
        pltpu.sync_copy(u_hbm.at[pl.ds(row0, rows_per_w)], u_v)

        bufs = (buf_a, buf_b)
        sems = (sem_a, sem_b)
        base_word = row0 * np1

        def chunk_src(ci):
            return pi_hbm.at[pl.ds(base_word + ci * chunk_words, chunk_words)]

        pltpu.async_copy(chunk_src(0), bufs[0], sems[0])

        lane = lax.iota(jnp.int32, _LANES)

        @pl.loop(0, n_chunks, step=2)
        def _chunk_loop(ci):
            for b in range(2):
                cur = ci + b

                @pl.when(cur + 1 < n_chunks)
                def _start_next():
                    pltpu.async_copy(chunk_src(cur + 1), bufs[1 - b],
                                     sems[1 - b])

                pltpu.make_async_copy(chunk_src(cur), bufs[b], sems[b]).wait()
                buf = bufs[b]

                @pl.loop(0, groups_per_chunk, step=_IL)
                def _group_loop(g):
                    idxs, us, accs, cnts = [], [], [], []
                    for q in range(_IL):
                        out_base = cur * chunk + (g + q) * _LANES
                        us.append(u_v[pl.ds(out_base, _LANES)])
                        idxs.append(((g + q) * _LANES + lane) * np1)
                        accs.append(jnp.zeros((_LANES,), jnp.float32))
                        cnts.append(jnp.zeros((_LANES,), jnp.int32))
                    for k in range(np1):
                        for q in range(_IL):
                            v = plsc.load_gather(buf, [idxs[q] + k])
                            accs[q] = accs[q] + v
                            cnts[q] = jnp.where(us[q] > accs[q],
                                                cnts[q] + 1, cnts[q])
                    for q in range(_IL):
                        out_base = cur * chunk + (g + q) * _LANES
                        out_v[pl.ds(out_base, _LANES)] = cnts[q]

        pltpu.sync_copy(out_v, out_hbm.at[pl.ds(row0, rows_per_w)])

    return sc_count


def kernel(pi_vectors):
    i_dim, m_dim, np1 = pi_vectors.shape
    rows = i_dim * m_dim
    u = _u_thresholds(i_dim, m_dim)
    pi_flat = pi_vectors.reshape(rows * np1)
    out = _build_sc_call(rows, np1)(pi_flat, u)
    return out.reshape(i_dim, m_dim)


# speedup vs baseline: 1.1159x; 1.1159x over previous
import functools

import numpy as np
import jax
import jax.numpy as jnp
from jax import lax
from jax.experimental import pallas as pl
from jax.experimental.pallas import tpu as pltpu
from jax.experimental.pallas import tpu_sc as plsc

_NUM_CORES = 2
_NUM_SUBCORES = 16
_LANES = 16
_NW = _NUM_CORES * _NUM_SUBCORES
_IL = 4


def _u_thresholds(i_dim: int, m_dim: int) -> jax.Array:
    u = jax.random.uniform(jax.random.key(42), (i_dim, m_dim, 1),
                           dtype=jnp.float32)
    return u.reshape(i_dim * m_dim)


@functools.lru_cache(maxsize=2)
def _build_sc_call(rows: int, np1: int):
    rows_per_w = rows // _NW
    chunk = 512
    assert rows_per_w % chunk == 0 and chunk % (_LANES * _IL) == 0
    n_chunks = rows_per_w // chunk
    assert n_chunks % 2 == 0
    chunk_words = chunk * np1
    groups_per_chunk = chunk // _LANES

    mesh = plsc.VectorSubcoreMesh(core_axis_name="c", subcore_axis_name="s")

    @functools.partial(
        pl.kernel,
        out_type=jax.ShapeDtypeStruct((rows,), jnp.int32),
        mesh=mesh,
        compiler_params=pltpu.CompilerParams(needs_layout_passes=False),
        scratch_types=[
            pltpu.VMEM((chunk_words,), jnp.float32),
            pltpu.VMEM((chunk_words,), jnp.float32),
            pltpu.VMEM((rows_per_w,), jnp.float32),
            pltpu.VMEM((rows_per_w,), jnp.int32),
            pltpu.SemaphoreType.DMA,
            pltpu.SemaphoreType.DMA,
        ],
    )
    def sc_count(pi_hbm, u_hbm, out_hbm, buf_a, buf_b, u_v, out_v,
                 sem_a, sem_b):
        wid = lax.axis_index("s") * _NUM_CORES + lax.axis_index("c")
        row0 = wid * rows_per_w
        pltpu.sync_copy(u_hbm.at[pl.ds(row0, rows_per_w)], u_v)

        bufs = (buf_a, buf_b)
        sems = (sem_a, sem_b)
        base_word = row0 * np1

        def chunk_src(ci):
            return pi_hbm.at[pl.ds(base_word + ci * chunk_words, chunk_words)]

        pltpu.async_copy(chunk_src(0), bufs[0], sems[0])

        lane = lax.iota(jnp.int32, _LANES)

        @pl.loop(0, n_chunks, step=2)
        def _chunk_loop(ci):
            for b in range(2):
                cur = ci + b

                @pl.when(cur + 1 < n_chunks)
                def _start_next():
                    pltpu.async_copy(chunk_src(cur + 1), bufs[1 - b],
                                     sems[1 - b])

                pltpu.make_async_copy(chunk_src(cur), bufs[b], sems[b]).wait()
                buf = bufs[b]

                @plsc.parallel_loop(0, groups_per_chunk, unroll=_IL)
                def _group_loop(g):
                    out_base = cur * chunk + g * _LANES
                    u_vec = u_v[pl.ds(out_base, _LANES)]
                    idx = (g * _LANES + lane) * np1
                    acc = jnp.zeros((_LANES,), jnp.float32)
                    cnt = jnp.zeros((_LANES,), jnp.int32)
                    for k in range(np1):
                        v = plsc.load_gather(buf, [idx + k])
                        acc = acc + v
                        cnt = jnp.where(u_vec > acc, cnt + 1, cnt)
                    out_v[pl.ds(out_base, _LANES)] = cnt

        pltpu.sync_copy(out_v, out_hbm.at[pl.ds(row0, rows_per_w)])

    return sc_count


def kernel(pi_vectors):
    i_dim, m_dim, np1 = pi_vectors.shape
    rows = i_dim * m_dim
    u = _u_thresholds(i_dim, m_dim)
    pi_flat = pi_vectors.reshape(rows * np1)
    out = _build_sc_call(rows, np1)(pi_flat, u)
    return out.reshape(i_dim, m_dim)
